# Initial kernel scaffold; baseline (speedup 1.0000x reference)
import jax
import jax.numpy as jnp
from jax.experimental import pallas as pl


def kernel(x, edge_index, edge_attr, edge_type, batch, beamforming, ris_phases, trajectory, node_W, node_b, edge_W, edge_b, leg_W1, leg_b1, leg_W2, leg_b2, eav_W1, eav_b1, eav_W2, eav_b2, att_W1, att_b1, att_W2, att_b2, upd_W, upd_b, ln_g, ln_b, ref_W1, ref_b1, ref_W2, ref_b2):
    raise NotImplementedError("write your pallas kernel here")


# speedup vs baseline: 2.1062x; 2.1062x over previous
import functools

import jax
import jax.numpy as jnp
from jax import lax
from jax.experimental import pallas as pl
from jax.experimental.pallas import tpu as pltpu
from jax.experimental.pallas import tpu_sc as plsc

N = 10000
E = 320000
H = 128
L = 2

NC = 2
NS = 16
NW = NC * NS
EPW = E // NW
ECH = 80
NCHK = EPW // ECH

CE = 512
CN = 1000
KE = 24
WG = 512

def _sc_mesh():
    return plsc.VectorSubcoreMesh(core_axis_name="c", subcore_axis_name="s",
                                  num_cores=NC, num_subcores=NS)


def _sc_gather_pair(h, dst_idx, src_idx):
    @functools.partial(
        pl.kernel,
        mesh=_sc_mesh(),
        out_type=(jax.ShapeDtypeStruct((E, H), jnp.float32),
                  jax.ShapeDtypeStruct((E, H), jnp.float32)),
        scratch_types=[
            pltpu.VMEM((NCHK, ECH), jnp.int32),
            pltpu.VMEM((NCHK, ECH), jnp.int32),
            pltpu.VMEM((ECH, H), jnp.float32),
            pltpu.VMEM((ECH, H), jnp.float32),
            pltpu.SemaphoreType.DMA,
            pltpu.SemaphoreType.DMA,
        ],
    )
    def k(h_hbm, dsti_hbm, srci_hbm, hd_hbm, hs_hbm,
          idxd, idxs, bufd, bufs, semd, sems):
        c = lax.axis_index("c")
        s = lax.axis_index("s")
        w = s * NC + c
        pltpu.sync_copy(dsti_hbm.at[w], idxd)
        pltpu.sync_copy(srci_hbm.at[w], idxs)
        base = w * EPW

        def body(j, carry):
            off = base + j * ECH
            cpd = pltpu.async_copy(h_hbm.at[idxd.at[j]], bufd, semd)
            cps = pltpu.async_copy(h_hbm.at[idxs.at[j]], bufs, sems)
            cpd.wait()
            pltpu.sync_copy(bufd, hd_hbm.at[pl.ds(off, ECH)])
            cps.wait()
            pltpu.sync_copy(bufs, hs_hbm.at[pl.ds(off, ECH)])
            return carry

        lax.fori_loop(0, NCHK, body, 0)

    return k(h, dst_idx, src_idx)


def _sc_scatter_add(msg, dst_idx, zeros_nh):
    @functools.partial(
        pl.kernel,
        mesh=_sc_mesh(),
        out_type=jax.ShapeDtypeStruct((NC, N, H), jnp.float32),
        scratch_types=[
            pltpu.VMEM((NCHK, ECH), jnp.int32),
            pltpu.VMEM((ECH, H), jnp.float32),
            pltpu.VMEM_SHARED((N, H), jnp.float32),
        ],
    )
    def k(msg_hbm, dsti_hbm, z_hbm, out_hbm, idxv, buf, accum):
        c = lax.axis_index("c")
        s = lax.axis_index("s")
        w = s * NC + c

        def zbody(t, carry):
            j = s + NS * t

            @pl.when(j < NCHK)
            def _():
                pltpu.sync_copy(z_hbm.at[pl.ds(j * ECH, ECH)],
                                accum.at[pl.ds(j * ECH, ECH)])
            return carry

        lax.fori_loop(0, (NCHK + NS - 1) // NS, zbody, 0)
        plsc.subcore_barrier()

        pltpu.sync_copy(dsti_hbm.at[w], idxv)
        base = w * EPW

        def body(j, carry):
            pltpu.sync_copy(msg_hbm.at[pl.ds(base + j * ECH, ECH)], buf)
            pltpu.sync_copy(buf, accum.at[idxv.at[j]], add=True)
            return carry

        lax.fori_loop(0, NCHK, body, 0)
        plsc.subcore_barrier()

        def obody(t, carry):
            j = s + NS * t

            @pl.when(j < NCHK)
            def _():
                pltpu.sync_copy(accum.at[pl.ds(j * ECH, ECH)],
                                out_hbm.at[c, pl.ds(j * ECH, ECH)])
            return carry

        lax.fori_loop(0, (NCHK + NS - 1) // NS, obody, 0)

    return k(msg, dst_idx, zeros_nh)


def _linear_body(x_ref, w_ref, b_ref, o_ref):
    o_ref[...] = jnp.dot(x_ref[...], w_ref[...],
                         preferred_element_type=jnp.float32) + b_ref[...]


def _tc_linear(x, w, b, bm):
    m, k = x.shape
    n = w.shape[1]
    return pl.pallas_call(
        _linear_body,
        grid=(m // bm,),
        in_specs=[
            pl.BlockSpec((bm, k), lambda i: (i, 0)),
            pl.BlockSpec((k, n), lambda i: (0, 0)),
            pl.BlockSpec((1, n), lambda i: (0, 0)),
        ],
        out_specs=pl.BlockSpec((bm, n), lambda i: (i, 0)),
        out_shape=jax.ShapeDtypeStruct((m, n), jnp.float32),
    )(x, w, b.reshape(1, n))


def _edge_body(hd_ref, hs_ref, ea_ref, w1p_ref, w1q_ref, w1e_ref,
               aw2_ref, ab2_ref, w2l_ref, w2e_ref, bl2_ref, be2_ref, o_ref):
    g = jnp.dot(hd_ref[...], w1p_ref[...], preferred_element_type=jnp.float32)
    g += jnp.dot(hs_ref[...], w1q_ref[...], preferred_element_type=jnp.float32)
    g += jnp.dot(ea_ref[...], w1e_ref[...], preferred_element_type=jnp.float32)
    g = jnp.maximum(g, 0.0)
    legr = g[:, 0:H]
    eavr = g[:, H:2 * H]
    attr = g[:, 2 * H:3 * H]
    etb = g[:, 3 * H:4 * H]
    att = jax.nn.sigmoid(
        jnp.sum(attr * aw2_ref[...], axis=1, keepdims=True) + ab2_ref[...])
    u = (1.0 - etb) * legr
    va = etb * att
    v = va * eavr
    msg = jnp.dot(u, w2l_ref[...], preferred_element_type=jnp.float32)
    msg += jnp.dot(v, w2e_ref[...], preferred_element_type=jnp.float32)
    msg += (1.0 - etb) * bl2_ref[...] + va * be2_ref[...]
    o_ref[...] = msg


def _tc_edge(hd, hs, eat, w1p, w1q, w1e, aw2, ab2, w2l, w2e, bl2, be2):
    return pl.pallas_call(
        _edge_body,
        grid=(E // CE,),
        in_specs=[
            pl.BlockSpec((CE, H), lambda i: (i, 0)),
            pl.BlockSpec((CE, H), lambda i: (i, 0)),
            pl.BlockSpec((CE, KE), lambda i: (i, 0)),
            pl.BlockSpec((H, WG), lambda i: (0, 0)),
            pl.BlockSpec((H, WG), lambda i: (0, 0)),
            pl.BlockSpec((KE, WG), lambda i: (0, 0)),
            pl.BlockSpec((1, H), lambda i: (0, 0)),
            pl.BlockSpec((1, 1), lambda i: (0, 0)),
            pl.BlockSpec((H, H), lambda i: (0, 0)),
            pl.BlockSpec((H, H), lambda i: (0, 0)),
            pl.BlockSpec((1, H), lambda i: (0, 0)),
            pl.BlockSpec((1, H), lambda i: (0, 0)),
        ],
        out_specs=pl.BlockSpec((CE, H), lambda i: (i, 0)),
        out_shape=jax.ShapeDtypeStruct((E, H), jnp.float32),
    )(hd, hs, eat, w1p, w1q, w1e, aw2, ab2, w2l, w2e, bl2, be2)


def _update_body(h_ref, p0_ref, p1_ref, w1_ref, w2_ref, b_ref, g_ref, be_ref,
                 hn_ref, gs_ref):
    i = pl.program_id(0)
    h = h_ref[...]
    aggr = p0_ref[...] + p1_ref[...]
    z = jnp.dot(h, w1_ref[...], preferred_element_type=jnp.float32)
    z += jnp.dot(aggr, w2_ref[...], preferred_element_type=jnp.float32)
    z += b_ref[...]
    mu = jnp.mean(z, axis=1, keepdims=True)
    d = z - mu
    var = jnp.mean(d * d, axis=1, keepdims=True)
    zn = d * jax.lax.rsqrt(var + 1e-5) * g_ref[...] + be_ref[...]
    hn = jnp.maximum(jnp.maximum(zn, 0.0) + h, 0.0)
    hn_ref[...] = hn

    @pl.when(i == 0)
    def _():
        gs_ref[...] = jnp.zeros_like(gs_ref)

    gs_ref[...] += jnp.sum(hn, axis=0, keepdims=True)


def _tc_update(h, p0, p1, uw1, uw2, ub, lng, lnb):
    return pl.pallas_call(
        _update_body,
        grid=(N // CN,),
        in_specs=[
            pl.BlockSpec((CN, H), lambda i: (i, 0)),
            pl.BlockSpec((CN, H), lambda i: (i, 0)),
            pl.BlockSpec((CN, H), lambda i: (i, 0)),
            pl.BlockSpec((H, H), lambda i: (0, 0)),
            pl.BlockSpec((H, H), lambda i: (0, 0)),
            pl.BlockSpec((1, H), lambda i: (0, 0)),
            pl.BlockSpec((1, H), lambda i: (0, 0)),
            pl.BlockSpec((1, H), lambda i: (0, 0)),
        ],
        out_specs=(pl.BlockSpec((CN, H), lambda i: (i, 0)),
                   pl.BlockSpec((1, H), lambda i: (0, 0))),
        out_shape=(jax.ShapeDtypeStruct((N, H), jnp.float32),
                   jax.ShapeDtypeStruct((1, H), jnp.float32)),
    )(h, p0, p1, uw1, uw2, ub.reshape(1, H), lng.reshape(1, H),
      lnb.reshape(1, H))


def _head_body(gs_ref, tr_ref, w1_ref, b1_ref, w2_ref, b2_ref,
               gr_ref, to_ref):
    gr = gs_ref[...] * (1.0 / N)
    r = jnp.maximum(
        jnp.dot(gr, w1_ref[...], preferred_element_type=jnp.float32)
        + b1_ref[...], 0.0)
    refn = jnp.dot(r, w2_ref[...], preferred_element_type=jnp.float32) \
        + b2_ref[...]
    gr_ref[...] = gr
    to_ref[...] = tr_ref[...] + refn


def _tc_head(gs, trajectory, rw1, rb1, rw2, rb2):
    out = rw2.shape[1]
    return pl.pallas_call(
        _head_body,
        out_shape=(jax.ShapeDtypeStruct((1, H), jnp.float32),
                   jax.ShapeDtypeStruct((1, out), jnp.float32)),
    )(gs, trajectory, rw1, rb1.reshape(1, H), rw2, rb2.reshape(1, out))


def kernel(x, edge_index, edge_attr, edge_type, batch, beamforming,
           ris_phases, trajectory, node_W, node_b, edge_W, edge_b,
           leg_W1, leg_b1, leg_W2, leg_b2, eav_W1, eav_b1, eav_W2, eav_b2,
           att_W1, att_b1, att_W2, att_b2, upd_W, upd_b, ln_g, ln_b,
           ref_W1, ref_b1, ref_W2, ref_b2):
    f32 = jnp.float32
    src = edge_index[0]
    dst = edge_index[1]
    dst_idx = dst.reshape(NW, NCHK, ECH)
    src_idx = src.reshape(NW, NCHK, ECH)
    et = edge_type.astype(f32)

    eat = jnp.concatenate(
        [edge_attr, et[:, None], jnp.ones((E, 1), f32),
         jnp.zeros((E, KE - 18), f32)], axis=1)
    zeros_nh = jnp.zeros((N, H), f32)

    h = _tc_linear(x, node_W, node_b, CN)

    for l in range(L):
        z64 = jnp.zeros((H, 64), f32)
        zH = jnp.zeros((H, H), f32)
        w1p = jnp.concatenate(
            [leg_W1[l][:H], eav_W1[l][:H], att_W1[l][:H], z64, zH], axis=1)
        w1q = jnp.concatenate(
            [leg_W1[l][H:2 * H], eav_W1[l][H:2 * H], att_W1[l][H:2 * H],
             z64, zH], axis=1)
        attr_rows = jnp.concatenate(
            [edge_W @ leg_W1[l][2 * H:], edge_W @ eav_W1[l][2 * H:],
             jnp.zeros((16, 2 * H), f32)], axis=1)
        et_row = jnp.concatenate(
            [jnp.zeros((1, 3 * H), f32), jnp.ones((1, H), f32)], axis=1)
        one_row = jnp.concatenate(
            [(edge_b @ leg_W1[l][2 * H:] + leg_b1[l])[None, :],
             (edge_b @ eav_W1[l][2 * H:] + eav_b1[l])[None, :],
             jnp.concatenate([att_b1[l], jnp.zeros((64,), f32)])[None, :],
             jnp.zeros((1, H), f32)], axis=1)
        w1e = jnp.concatenate(
            [attr_rows, et_row, one_row, jnp.zeros((KE - 18, WG), f32)],
            axis=0)
        aw2 = jnp.concatenate([att_W2[l][:, 0], jnp.zeros((64,), f32)])
        aw2 = aw2.reshape(1, H)
        ab2 = att_b2[l].reshape(1, 1)

        hd, hs = _sc_gather_pair(h, dst_idx, src_idx)
        msg = _tc_edge(hd, hs, eat, w1p, w1q, w1e, aw2, ab2,
                       leg_W2[l], eav_W2[l],
                       leg_b2[l].reshape(1, H), eav_b2[l].reshape(1, H))
        parts = _sc_scatter_add(msg, dst_idx, zeros_nh)
        h, gs = _tc_update(h, parts[0], parts[1], upd_W[l][:H],
                           upd_W[l][H:], upd_b[l], ln_g[l], ln_b[l])

    graph_repr, traj_out = _tc_head(gs, trajectory, ref_W1, ref_b1,
                                    ref_W2, ref_b2)
    return (beamforming, ris_phases, traj_out, graph_repr)


# speedup vs baseline: 2.3102x; 1.0968x over previous
import functools

import jax
import jax.numpy as jnp
from jax import lax
from jax.experimental import pallas as pl
from jax.experimental.pallas import tpu as pltpu
from jax.experimental.pallas import tpu_sc as plsc

N = 10000
E = 320000
H = 128
L = 2

NC = 2
NS = 16
NW = NC * NS
EPW = E // NW
ECH = 80
NCHK = EPW // ECH

CE = 512
CN = 1000
KE = 24
WG = 384

def _sc_mesh():
    return plsc.VectorSubcoreMesh(core_axis_name="c", subcore_axis_name="s",
                                  num_cores=NC, num_subcores=NS)


def _sc_gather_pair(h, dst_idx, src_idx):
    @functools.partial(
        pl.kernel,
        mesh=_sc_mesh(),
        out_type=(jax.ShapeDtypeStruct((E, H), jnp.float32),
                  jax.ShapeDtypeStruct((E, H), jnp.float32)),
        scratch_types=[
            pltpu.VMEM((NCHK, ECH), jnp.int32),
            pltpu.VMEM((NCHK, ECH), jnp.int32),
            pltpu.VMEM((ECH, H), jnp.float32),
            pltpu.VMEM((ECH, H), jnp.float32),
            pltpu.SemaphoreType.DMA,
            pltpu.SemaphoreType.DMA,
        ],
    )
    def k(h_hbm, dsti_hbm, srci_hbm, hd_hbm, hs_hbm,
          idxd, idxs, bufd, bufs, semd, sems):
        c = lax.axis_index("c")
        s = lax.axis_index("s")
        w = s * NC + c
        pltpu.sync_copy(dsti_hbm.at[w], idxd)
        pltpu.sync_copy(srci_hbm.at[w], idxs)
        base = w * EPW

        def body(j, carry):
            off = base + j * ECH
            cpd = pltpu.async_copy(h_hbm.at[idxd.at[j]], bufd, semd)
            cps = pltpu.async_copy(h_hbm.at[idxs.at[j]], bufs, sems)
            cpd.wait()
            pltpu.sync_copy(bufd, hd_hbm.at[pl.ds(off, ECH)])
            cps.wait()
            pltpu.sync_copy(bufs, hs_hbm.at[pl.ds(off, ECH)])
            return carry

        lax.fori_loop(0, NCHK, body, 0)

    return k(h, dst_idx, src_idx)


def _sc_scatter_add(msg, dst_idx, zeros_nh):
    @functools.partial(
        pl.kernel,
        mesh=_sc_mesh(),
        out_type=jax.ShapeDtypeStruct((NC, N, H), jnp.float32),
        scratch_types=[
            pltpu.VMEM((NCHK, ECH), jnp.int32),
            pltpu.VMEM((ECH, H), jnp.float32),
            pltpu.VMEM_SHARED((N, H), jnp.float32),
        ],
    )
    def k(msg_hbm, dsti_hbm, z_hbm, out_hbm, idxv, buf, accum):
        c = lax.axis_index("c")
        s = lax.axis_index("s")
        w = s * NC + c

        def zbody(t, carry):
            j = s + NS * t

            @pl.when(j < NCHK)
            def _():
                pltpu.sync_copy(z_hbm.at[pl.ds(j * ECH, ECH)],
                                accum.at[pl.ds(j * ECH, ECH)])
            return carry

        lax.fori_loop(0, (NCHK + NS - 1) // NS, zbody, 0)
        plsc.subcore_barrier()

        pltpu.sync_copy(dsti_hbm.at[w], idxv)
        base = w * EPW

        def body(j, carry):
            pltpu.sync_copy(msg_hbm.at[pl.ds(base + j * ECH, ECH)], buf)
            pltpu.sync_copy(buf, accum.at[idxv.at[j]], add=True)
            return carry

        lax.fori_loop(0, NCHK, body, 0)
        plsc.subcore_barrier()

        def obody(t, carry):
            j = s + NS * t

            @pl.when(j < NCHK)
            def _():
                pltpu.sync_copy(accum.at[pl.ds(j * ECH, ECH)],
                                out_hbm.at[c, pl.ds(j * ECH, ECH)])
            return carry

        lax.fori_loop(0, (NCHK + NS - 1) // NS, obody, 0)

    return k(msg, dst_idx, zeros_nh)


def _linear_body(x_ref, w_ref, b_ref, o_ref):
    o_ref[...] = jnp.dot(x_ref[...], w_ref[...],
                         preferred_element_type=jnp.float32) + b_ref[...]


def _tc_linear(x, w, b, bm):
    m, k = x.shape
    n = w.shape[1]
    return pl.pallas_call(
        _linear_body,
        grid=(m // bm,),
        in_specs=[
            pl.BlockSpec((bm, k), lambda i: (i, 0)),
            pl.BlockSpec((k, n), lambda i: (0, 0)),
            pl.BlockSpec((1, n), lambda i: (0, 0)),
        ],
        out_specs=pl.BlockSpec((bm, n), lambda i: (i, 0)),
        out_shape=jax.ShapeDtypeStruct((m, n), jnp.float32),
    )(x, w, b.reshape(1, n))


def _edge_body(hd_ref, hs_ref, ea_ref, w1p_ref, w1q_ref, w1e_ref,
               aw2_ref, esel_ref, ab2_ref, w2l_ref, w2e_ref,
               bl2_ref, be2_ref, o_ref):
    bf16 = jnp.bfloat16
    g = jnp.dot(hd_ref[...].astype(bf16), w1p_ref[...],
                preferred_element_type=jnp.float32)
    g += jnp.dot(hs_ref[...].astype(bf16), w1q_ref[...],
                 preferred_element_type=jnp.float32)
    g += jax.lax.dot_general(ea_ref[...].astype(bf16), w1e_ref[...],
                             (((0,), (0,)), ((), ())),
                             preferred_element_type=jnp.float32)
    g = jnp.maximum(g, 0.0)
    legr = g[:, 0:H]
    eavr = g[:, H:2 * H]
    grp = g[:, 2 * H:3 * H]
    att = jax.nn.sigmoid(
        jnp.sum(grp * aw2_ref[...], axis=1, keepdims=True) + ab2_ref[...])
    etc = jnp.sum(grp * esel_ref[...], axis=1, keepdims=True)
    u = (1.0 - etc) * legr
    va = etc * att
    v = va * eavr
    msg = jnp.dot(u.astype(bf16), w2l_ref[...],
                  preferred_element_type=jnp.float32)
    msg += jnp.dot(v.astype(bf16), w2e_ref[...],
                   preferred_element_type=jnp.float32)
    msg += (1.0 - etc) * bl2_ref[...] + va * be2_ref[...]
    o_ref[...] = msg


def _tc_edge(hd, hs, eatT, w1p, w1q, w1e, aw2, esel, ab2, w2l, w2e, bl2, be2):
    return pl.pallas_call(
        _edge_body,
        grid=(E // CE,),
        in_specs=[
            pl.BlockSpec((CE, H), lambda i: (i, 0)),
            pl.BlockSpec((CE, H), lambda i: (i, 0)),
            pl.BlockSpec((KE, CE), lambda i: (0, i)),
            pl.BlockSpec((H, WG), lambda i: (0, 0)),
            pl.BlockSpec((H, WG), lambda i: (0, 0)),
            pl.BlockSpec((KE, WG), lambda i: (0, 0)),
            pl.BlockSpec((1, H), lambda i: (0, 0)),
            pl.BlockSpec((1, H), lambda i: (0, 0)),
            pl.BlockSpec((1, 1), lambda i: (0, 0)),
            pl.BlockSpec((H, H), lambda i: (0, 0)),
            pl.BlockSpec((H, H), lambda i: (0, 0)),
            pl.BlockSpec((1, H), lambda i: (0, 0)),
            pl.BlockSpec((1, H), lambda i: (0, 0)),
        ],
        out_specs=pl.BlockSpec((CE, H), lambda i: (i, 0)),
        out_shape=jax.ShapeDtypeStruct((E, H), jnp.float32),
    )(hd, hs, eatT, w1p, w1q, w1e, aw2, esel, ab2, w2l, w2e, bl2, be2)


def _update_body(h_ref, p0_ref, p1_ref, w1_ref, w2_ref, b_ref, g_ref, be_ref,
                 hn_ref, gs_ref):
    i = pl.program_id(0)
    h = h_ref[...]
    aggr = p0_ref[...] + p1_ref[...]
    z = jnp.dot(h, w1_ref[...], preferred_element_type=jnp.float32)
    z += jnp.dot(aggr, w2_ref[...], preferred_element_type=jnp.float32)
    z += b_ref[...]
    mu = jnp.mean(z, axis=1, keepdims=True)
    d = z - mu
    var = jnp.mean(d * d, axis=1, keepdims=True)
    zn = d * jax.lax.rsqrt(var + 1e-5) * g_ref[...] + be_ref[...]
    hn = jnp.maximum(jnp.maximum(zn, 0.0) + h, 0.0)
    hn_ref[...] = hn

    @pl.when(i == 0)
    def _():
        gs_ref[...] = jnp.zeros_like(gs_ref)

    gs_ref[...] += jnp.sum(hn, axis=0, keepdims=True)


def _tc_update(h, p0, p1, uw1, uw2, ub, lng, lnb):
    return pl.pallas_call(
        _update_body,
        grid=(N // CN,),
        in_specs=[
            pl.BlockSpec((CN, H), lambda i: (i, 0)),
            pl.BlockSpec((CN, H), lambda i: (i, 0)),
            pl.BlockSpec((CN, H), lambda i: (i, 0)),
            pl.BlockSpec((H, H), lambda i: (0, 0)),
            pl.BlockSpec((H, H), lambda i: (0, 0)),
            pl.BlockSpec((1, H), lambda i: (0, 0)),
            pl.BlockSpec((1, H), lambda i: (0, 0)),
            pl.BlockSpec((1, H), lambda i: (0, 0)),
        ],
        out_specs=(pl.BlockSpec((CN, H), lambda i: (i, 0)),
                   pl.BlockSpec((1, H), lambda i: (0, 0))),
        out_shape=(jax.ShapeDtypeStruct((N, H), jnp.float32),
                   jax.ShapeDtypeStruct((1, H), jnp.float32)),
    )(h, p0, p1, uw1, uw2, ub.reshape(1, H), lng.reshape(1, H),
      lnb.reshape(1, H))


def _head_body(gs_ref, tr_ref, w1_ref, b1_ref, w2_ref, b2_ref,
               gr_ref, to_ref):
    gr = gs_ref[...] * (1.0 / N)
    r = jnp.maximum(
        jnp.dot(gr, w1_ref[...], preferred_element_type=jnp.float32)
        + b1_ref[...], 0.0)
    refn = jnp.dot(r, w2_ref[...], preferred_element_type=jnp.float32) \
        + b2_ref[...]
    gr_ref[...] = gr
    to_ref[...] = tr_ref[...] + refn


def _tc_head(gs, trajectory, rw1, rb1, rw2, rb2):
    out = rw2.shape[1]
    return pl.pallas_call(
        _head_body,
        out_shape=(jax.ShapeDtypeStruct((1, H), jnp.float32),
                   jax.ShapeDtypeStruct((1, out), jnp.float32)),
    )(gs, trajectory, rw1, rb1.reshape(1, H), rw2, rb2.reshape(1, out))


def kernel(x, edge_index, edge_attr, edge_type, batch, beamforming,
           ris_phases, trajectory, node_W, node_b, edge_W, edge_b,
           leg_W1, leg_b1, leg_W2, leg_b2, eav_W1, eav_b1, eav_W2, eav_b2,
           att_W1, att_b1, att_W2, att_b2, upd_W, upd_b, ln_g, ln_b,
           ref_W1, ref_b1, ref_W2, ref_b2):
    f32 = jnp.float32
    src = edge_index[0]
    dst = edge_index[1]
    dst_idx = dst.reshape(NW, NCHK, ECH)
    src_idx = src.reshape(NW, NCHK, ECH)
    et = edge_type.astype(f32)

    eatT = jnp.concatenate(
        [edge_attr.T, et[None, :], jnp.ones((1, E), f32),
         jnp.zeros((KE - 18, E), f32)], axis=0)
    zeros_nh = jnp.zeros((N, H), f32)

    h = _tc_linear(x, node_W, node_b, CN)

    bf16 = jnp.bfloat16
    for l in range(L):
        z64 = jnp.zeros((H, 64), f32)
        w1p = jnp.concatenate(
            [leg_W1[l][:H], eav_W1[l][:H], att_W1[l][:H], z64],
            axis=1).astype(bf16)
        w1q = jnp.concatenate(
            [leg_W1[l][H:2 * H], eav_W1[l][H:2 * H], att_W1[l][H:2 * H],
             z64], axis=1).astype(bf16)
        attr_rows = jnp.concatenate(
            [edge_W @ leg_W1[l][2 * H:], edge_W @ eav_W1[l][2 * H:],
             jnp.zeros((16, H), f32)], axis=1)
        et_row = jnp.concatenate(
            [jnp.zeros((1, 2 * H + 64), f32), jnp.ones((1, 64), f32)],
            axis=1)
        one_row = jnp.concatenate(
            [(edge_b @ leg_W1[l][2 * H:] + leg_b1[l])[None, :],
             (edge_b @ eav_W1[l][2 * H:] + eav_b1[l])[None, :],
             att_b1[l][None, :], jnp.zeros((1, 64), f32)], axis=1)
        w1e = jnp.concatenate(
            [attr_rows, et_row, one_row, jnp.zeros((KE - 18, WG), f32)],
            axis=0).astype(bf16)
        aw2 = jnp.concatenate([att_W2[l][:, 0], jnp.zeros((64,), f32)])
        aw2 = aw2.reshape(1, H)
        esel = jnp.concatenate([jnp.zeros((64,), f32),
                                jnp.full((64,), 1.0 / 64, f32)]).reshape(1, H)
        ab2 = att_b2[l].reshape(1, 1)

        hd, hs = _sc_gather_pair(h, dst_idx, src_idx)
        msg = _tc_edge(hd, hs, eatT, w1p, w1q, w1e, aw2, esel, ab2,
                       leg_W2[l].astype(bf16), eav_W2[l].astype(bf16),
                       leg_b2[l].reshape(1, H), eav_b2[l].reshape(1, H))
        parts = _sc_scatter_add(msg, dst_idx, zeros_nh)
        h, gs = _tc_update(h, parts[0], parts[1], upd_W[l][:H],
                           upd_W[l][H:], upd_b[l], ln_g[l], ln_b[l])

    graph_repr, traj_out = _tc_head(gs, trajectory, ref_W1, ref_b1,
                                    ref_W2, ref_b2)
    return (beamforming, ris_phases, traj_out, graph_repr)
